# baseline (speedup 1.0000x reference)
import jax
import jax.numpy as jnp
from jax.experimental import pallas as pl
from jax.experimental.pallas import tpu as pltpu

_B, _DIN, _HID, _CODE, _K = 4096, 1024, 1024, 256, 1024
_BM = 512
_NBLK = _B // _BM

_DEF = jax.lax.Precision.DEFAULT


def _dot(a, b, prec=_DEF):
    return jax.lax.dot_general(
        a, b, (((1,), (0,)), ((), ())),
        precision=prec, preferred_element_type=jnp.float32)


def _body(x_ref, We1_ref, be1_ref, We2_ref, be2_ref, We3_ref, be3_ref,
          Wd1_ref, bd1_ref, Wd2_ref, bd2_ref, Wd3_ref, bd3_ref, cb_ref,
          xt_ref, ze_ref, zq_ref,
          we1_s, we2_s, we3_s, wd1_s, wd2_s, wd3_s, cbh_s, cbl_s, csq_s,
          codes_s, ze_s, zq_s):
    bf = jnp.bfloat16
    i = pl.program_id(0)
    wr = jax.lax.rem(i, 2)
    rd = jax.lax.rem(i + 1, 2)

    @pl.when(i == 0)
    def _init():
        we1_s[...] = We1_ref[...].astype(bf)
        we2_s[...] = We2_ref[...].astype(bf)
        we3_s[...] = We3_ref[...].astype(bf)
        wd1_s[...] = Wd1_ref[...].astype(bf)
        wd2_s[...] = Wd2_ref[...].astype(bf)
        wd3_s[...] = Wd3_ref[...].astype(bf)
        cbf = cb_ref[...]
        hi = cbf.astype(bf)
        cbh_s[...] = hi
        cbl_s[...] = (cbf - hi.astype(jnp.float32)).astype(bf)
        csq_s[...] = jnp.sum(cbf * cbf, axis=1)[None, :]

    x = x_ref[...]
    h = jnp.maximum(_dot(x, we1_s[...]) + be1_ref[...], 0.0)
    h = jnp.maximum(_dot(h, we2_s[...]) + be2_ref[...], 0.0)
    z_e = _dot(h, we3_s[...]) + be3_ref[...]
    ze_s[wr] = z_e

    zc = jax.lax.dot_general(
        z_e, cbh_s[...], (((1,), (1,)), ((), ())),
        precision=_DEF, preferred_element_type=jnp.float32)

    codes = codes_s[rd]
    hd = jnp.maximum(_dot(codes, wd1_s[...]) + bd1_ref[...], 0.0)

    z_sqr = jnp.sum(z_e * z_e, axis=1, keepdims=True)
    dist = z_sqr + csq_s[...] - 2.0 * zc
    m = jnp.min(dist, axis=1, keepdims=True)

    hd = jnp.maximum(_dot(hd.astype(bf), wd2_s[...]) + bd2_ref[...], 0.0)

    iota = jax.lax.broadcasted_iota(jnp.int32, dist.shape, 1)
    idx = jnp.min(jnp.where(dist == m, iota, _K), axis=1)
    onehot = (iota == idx[:, None]).astype(bf)

    xt_ref[...] = _dot(hd.astype(bf), wd3_s[...]) + bd3_ref[...]
    ze_ref[...] = ze_s[rd]
    zq_ref[...] = zq_s[rd]

    codes_hi = _dot(onehot, cbh_s[...])
    codes_s[wr] = codes_hi.astype(bf)
    zq_s[wr] = codes_hi + _dot(onehot, cbl_s[...])


def _full(shape):
    return pl.BlockSpec(shape, lambda i: (0, 0))


def kernel(x, We1, be1, We2, be2, We3, be3,
           Wd1, bd1, Wd2, bd2, Wd3, bd3, codebook):
    grid = (_NBLK + 1,)
    out_shape = (
        jax.ShapeDtypeStruct((_B, _DIN), jnp.float32),
        jax.ShapeDtypeStruct((_B, _CODE), jnp.float32),
        jax.ShapeDtypeStruct((_B, _CODE), jnp.float32),
    )
    in_specs = [
        pl.BlockSpec((_BM, _DIN), lambda i: (jnp.minimum(i, _NBLK - 1), 0)),
        _full((_DIN, _HID)), _full((1, _HID)),
        _full((_HID, _HID)), _full((1, _HID)),
        _full((_HID, _CODE)), _full((1, _CODE)),
        _full((_CODE, _HID)), _full((1, _HID)),
        _full((_HID, _HID)), _full((1, _HID)),
        _full((_HID, _DIN)), _full((1, _DIN)),
        _full((_K, _CODE)),
    ]
    _oidx = lambda i: (jnp.maximum(i - 1, 0), 0)
    out_specs = (
        pl.BlockSpec((_BM, _DIN), _oidx),
        pl.BlockSpec((_BM, _CODE), _oidx),
        pl.BlockSpec((_BM, _CODE), _oidx),
    )
    bf = jnp.bfloat16
    scratch_shapes = [
        pltpu.VMEM((_DIN, _HID), bf),
        pltpu.VMEM((_HID, _HID), bf),
        pltpu.VMEM((_HID, _CODE), bf),
        pltpu.VMEM((_CODE, _HID), bf),
        pltpu.VMEM((_HID, _HID), bf),
        pltpu.VMEM((_HID, _DIN), bf),
        pltpu.VMEM((_K, _CODE), bf),
        pltpu.VMEM((_K, _CODE), bf),
        pltpu.VMEM((1, _K), jnp.float32),
        pltpu.VMEM((2, _BM, _CODE), bf),
        pltpu.VMEM((2, _BM, _CODE), jnp.float32),
        pltpu.VMEM((2, _BM, _CODE), jnp.float32),
    ]
    xt, ze, zq = pl.pallas_call(
        _body,
        grid=grid,
        in_specs=in_specs,
        out_specs=out_specs,
        out_shape=out_shape,
        scratch_shapes=scratch_shapes,
        compiler_params=pltpu.CompilerParams(
            dimension_semantics=("arbitrary",),
        ),
    )(x, We1, be1.reshape(1, -1), We2, be2.reshape(1, -1),
      We3, be3.reshape(1, -1), Wd1, bd1.reshape(1, -1),
      Wd2, bd2.reshape(1, -1), Wd3, bd3.reshape(1, -1), codebook)
    return xt, ze, zq


# speedup vs baseline: 1.0028x; 1.0028x over previous
import jax
import jax.numpy as jnp
from jax.experimental import pallas as pl
from jax.experimental.pallas import tpu as pltpu

_B, _DIN, _HID, _CODE, _K = 4096, 1024, 1024, 256, 1024
_BM = 512

_DEF = jax.lax.Precision.DEFAULT


def _dot(a, b, prec=_DEF):
    return jax.lax.dot_general(
        a, b, (((1,), (0,)), ((), ())),
        precision=prec, preferred_element_type=jnp.float32)


def _body(x_ref, We1_ref, be1_ref, We2_ref, be2_ref, We3_ref, be3_ref,
          Wd1_ref, bd1_ref, Wd2_ref, bd2_ref, Wd3_ref, bd3_ref, cb_ref,
          xt_ref, ze_ref, zq_ref,
          we1_s, we2_s, we3_s, wd1_s, wd2_s, wd3_s, cbh_s, cbl_s, csq_s,
          stage_s, sem):
    bf = jnp.bfloat16

    @pl.when(pl.program_id(0) == 0)
    def _init():
        def _stage(src_ref, rows, cols):
            cp = pltpu.make_async_copy(src_ref, stage_s.at[:rows, :cols], sem)
            cp.start()
            cp.wait()
            return stage_s[:rows, :cols]

        we1_s[...] = _stage(We1_ref, _DIN, _HID).astype(bf)
        we2_s[...] = _stage(We2_ref, _HID, _HID).astype(bf)
        we3_s[...] = _stage(We3_ref, _HID, _CODE).astype(bf)
        wd1_s[...] = _stage(Wd1_ref, _CODE, _HID).astype(bf)
        wd2_s[...] = _stage(Wd2_ref, _HID, _HID).astype(bf)
        wd3_s[...] = _stage(Wd3_ref, _HID, _DIN).astype(bf)
        cbf = _stage(cb_ref, _K, _CODE)
        hi = cbf.astype(bf)
        cbh_s[...] = hi
        cbl_s[...] = (cbf - hi.astype(jnp.float32)).astype(bf)
        csq_s[...] = jnp.sum(cbf * cbf, axis=1)[None, :]

    x = x_ref[...]
    h = jnp.maximum(_dot(x, we1_s[...]) + be1_ref[...], 0.0)
    h = jnp.maximum(_dot(h, we2_s[...]) + be2_ref[...], 0.0)
    z_e = _dot(h, we3_s[...]) + be3_ref[...]
    ze_ref[...] = z_e

    z_sqr = jnp.sum(z_e * z_e, axis=1, keepdims=True)
    zc = jax.lax.dot_general(
        z_e, cbh_s[...], (((1,), (1,)), ((), ())),
        precision=_DEF, preferred_element_type=jnp.float32)
    dist = z_sqr + csq_s[...] - 2.0 * zc

    m = jnp.min(dist, axis=1, keepdims=True)
    iota = jax.lax.broadcasted_iota(jnp.int32, dist.shape, 1)
    idx = jnp.min(jnp.where(dist == m, iota, _K), axis=1)
    onehot = (iota == idx[:, None]).astype(bf)
    codes_hi = _dot(onehot, cbh_s[...])
    zq_ref[...] = codes_hi + _dot(onehot, cbl_s[...])

    h = jnp.maximum(_dot(codes_hi.astype(bf), wd1_s[...]) + bd1_ref[...], 0.0)
    h = jnp.maximum(_dot(h.astype(bf), wd2_s[...]) + bd2_ref[...], 0.0)
    xt_ref[...] = _dot(h.astype(bf), wd3_s[...]) + bd3_ref[...]


def _hbm():
    return pl.BlockSpec(memory_space=pltpu.MemorySpace.HBM)


def _full(shape):
    return pl.BlockSpec(shape, lambda i: (0, 0))


def kernel(x, We1, be1, We2, be2, We3, be3,
           Wd1, bd1, Wd2, bd2, Wd3, bd3, codebook):
    grid = (_B // _BM,)
    out_shape = (
        jax.ShapeDtypeStruct((_B, _DIN), jnp.float32),
        jax.ShapeDtypeStruct((_B, _CODE), jnp.float32),
        jax.ShapeDtypeStruct((_B, _CODE), jnp.float32),
    )
    in_specs = [
        pl.BlockSpec((_BM, _DIN), lambda i: (i, 0)),
        _hbm(), _full((1, _HID)),
        _hbm(), _full((1, _HID)),
        _hbm(), _full((1, _CODE)),
        _hbm(), _full((1, _HID)),
        _hbm(), _full((1, _HID)),
        _hbm(), _full((1, _DIN)),
        _hbm(),
    ]
    out_specs = (
        pl.BlockSpec((_BM, _DIN), lambda i: (i, 0)),
        pl.BlockSpec((_BM, _CODE), lambda i: (i, 0)),
        pl.BlockSpec((_BM, _CODE), lambda i: (i, 0)),
    )
    bf = jnp.bfloat16
    scratch_shapes = [
        pltpu.VMEM((_DIN, _HID), bf),
        pltpu.VMEM((_HID, _HID), bf),
        pltpu.VMEM((_HID, _CODE), bf),
        pltpu.VMEM((_CODE, _HID), bf),
        pltpu.VMEM((_HID, _HID), bf),
        pltpu.VMEM((_HID, _DIN), bf),
        pltpu.VMEM((_K, _CODE), bf),
        pltpu.VMEM((_K, _CODE), bf),
        pltpu.VMEM((1, _K), jnp.float32),
        pltpu.VMEM((_DIN, _HID), jnp.float32),
        pltpu.SemaphoreType.DMA,
    ]
    xt, ze, zq = pl.pallas_call(
        _body,
        grid=grid,
        in_specs=in_specs,
        out_specs=out_specs,
        out_shape=out_shape,
        scratch_shapes=scratch_shapes,
        compiler_params=pltpu.CompilerParams(
            dimension_semantics=("arbitrary",),
        ),
    )(x, We1, be1.reshape(1, -1), We2, be2.reshape(1, -1),
      We3, be3.reshape(1, -1), Wd1, bd1.reshape(1, -1),
      Wd2, bd2.reshape(1, -1), Wd3, bd3.reshape(1, -1), codebook)
    return xt, ze, zq


# speedup vs baseline: 1.1046x; 1.1014x over previous
import jax
import jax.numpy as jnp
from jax.experimental import pallas as pl
from jax.experimental.pallas import tpu as pltpu

_B, _DIN, _HID, _CODE, _K = 4096, 1024, 1024, 256, 1024
_BM = 512

_DEF = jax.lax.Precision.DEFAULT


def _dot(a, b, prec=_DEF):
    return jax.lax.dot_general(
        a, b, (((1,), (0,)), ((), ())),
        precision=prec, preferred_element_type=jnp.float32)


def _body(x_ref, We1_ref, be1_ref, We2_ref, be2_ref, We3_ref, be3_ref,
          Wd1_ref, bd1_ref, Wd2_ref, bd2_ref, Wd3_ref, bd3_ref, cb_ref,
          xt_ref, ze_ref, zq_ref,
          we1_s, we2_s, we3_s, wd1_s, wd2_s, wd3_s, cbh_s, cbl_s, csq_s):
    bf = jnp.bfloat16

    @pl.when(pl.program_id(0) == 0)
    def _init():
        we1_s[...] = We1_ref[...].astype(bf)
        we2_s[...] = We2_ref[...].astype(bf)
        we3_s[...] = We3_ref[...].astype(bf)
        wd1_s[...] = Wd1_ref[...].astype(bf)
        wd2_s[...] = Wd2_ref[...].astype(bf)
        wd3_s[...] = Wd3_ref[...].astype(bf)
        cbf = cb_ref[...]
        hi = cbf.astype(bf)
        cbh_s[...] = hi
        cbl_s[...] = (cbf - hi.astype(jnp.float32)).astype(bf)
        csq_s[...] = jnp.sum(cbf * cbf, axis=1)[None, :]

    x = x_ref[...]
    h = jnp.maximum(_dot(x, we1_s[...]) + be1_ref[...], 0.0)
    h = jnp.maximum(_dot(h, we2_s[...]) + be2_ref[...], 0.0)
    z_e = _dot(h, we3_s[...]) + be3_ref[...]
    ze_ref[...] = z_e

    z_sqr = jnp.sum(z_e * z_e, axis=1, keepdims=True)
    zc = jax.lax.dot_general(
        z_e, cbh_s[...], (((1,), (1,)), ((), ())),
        precision=_DEF, preferred_element_type=jnp.float32)
    dist = z_sqr + csq_s[...] - 2.0 * zc

    m = jnp.min(dist, axis=1, keepdims=True)
    iota = jax.lax.broadcasted_iota(jnp.int32, dist.shape, 1)
    idx = jnp.min(jnp.where(dist == m, iota, _K), axis=1)
    onehot = (iota == idx[:, None]).astype(bf)
    codes_hi = _dot(onehot, cbh_s[...])
    zq_ref[...] = codes_hi + _dot(onehot, cbl_s[...])

    h = jnp.maximum(_dot(codes_hi.astype(bf), wd1_s[...]) + bd1_ref[...], 0.0)
    h = jnp.maximum(_dot(h.astype(bf), wd2_s[...]) + bd2_ref[...], 0.0)
    xt_ref[...] = _dot(h.astype(bf), wd3_s[...]) + bd3_ref[...]


def _full(shape):
    return pl.BlockSpec(shape, lambda i: (0, 0))


def kernel(x, We1, be1, We2, be2, We3, be3,
           Wd1, bd1, Wd2, bd2, Wd3, bd3, codebook):
    grid = (_B // _BM,)
    out_shape = (
        jax.ShapeDtypeStruct((_B, _DIN), jnp.float32),
        jax.ShapeDtypeStruct((_B, _CODE), jnp.float32),
        jax.ShapeDtypeStruct((_B, _CODE), jnp.float32),
    )
    in_specs = [
        pl.BlockSpec((_BM, _DIN), lambda i: (i, 0)),
        _full((_DIN, _HID)), _full((1, _HID)),
        _full((_HID, _HID)), _full((1, _HID)),
        _full((_HID, _CODE)), _full((1, _CODE)),
        _full((_CODE, _HID)), _full((1, _HID)),
        _full((_HID, _HID)), _full((1, _HID)),
        _full((_HID, _DIN)), _full((1, _DIN)),
        _full((_K, _CODE)),
    ]
    out_specs = (
        pl.BlockSpec((_BM, _DIN), lambda i: (i, 0)),
        pl.BlockSpec((_BM, _CODE), lambda i: (i, 0)),
        pl.BlockSpec((_BM, _CODE), lambda i: (i, 0)),
    )
    bf = jnp.bfloat16
    scratch_shapes = [
        pltpu.VMEM((_DIN, _HID), bf),
        pltpu.VMEM((_HID, _HID), bf),
        pltpu.VMEM((_HID, _CODE), bf),
        pltpu.VMEM((_CODE, _HID), bf),
        pltpu.VMEM((_HID, _HID), bf),
        pltpu.VMEM((_HID, _DIN), bf),
        pltpu.VMEM((_K, _CODE), bf),
        pltpu.VMEM((_K, _CODE), bf),
        pltpu.VMEM((1, _K), jnp.float32),
    ]
    xt, ze, zq = pl.pallas_call(
        _body,
        grid=grid,
        in_specs=in_specs,
        out_specs=out_specs,
        out_shape=out_shape,
        scratch_shapes=scratch_shapes,
        compiler_params=pltpu.CompilerParams(
            dimension_semantics=("arbitrary",),
        ),
    )(x, We1, be1.reshape(1, -1), We2, be2.reshape(1, -1),
      We3, be3.reshape(1, -1), Wd1, bd1.reshape(1, -1),
      Wd2, bd2.reshape(1, -1), Wd3, bd3.reshape(1, -1), codebook)
    return xt, ze, zq
